# Initial kernel scaffold; baseline (speedup 1.0000x reference)
import jax
import jax.numpy as jnp
from jax.experimental import pallas as pl


def kernel(node_idx, edge_idx, node_table, edge_table):
    raise NotImplementedError("write your pallas kernel here")


# speedup vs baseline: 4.2025x; 4.2025x over previous
import functools
import math

import jax
import jax.numpy as jnp
import numpy as np
from jax import lax
from jax.experimental import pallas as pl
from jax.experimental.pallas import tpu as pltpu
from jax.experimental.pallas import tpu_sc as plsc

S = 200
B = 4096
D = 64
L = 16

_info = plsc.get_sparse_core_info()
NC = _info.num_cores
NS = _info.num_subcores
NW = NC * NS
BPW = B // NW


def _positional_encoding_np(seq_len: int, d_model: int) -> np.ndarray:
    position = np.arange(seq_len, dtype=np.float32)[:, None]
    div_term = np.exp(
        np.arange(0, d_model, 2, dtype=np.float32) * (-math.log(10000.0) / d_model)
    )
    pe = np.zeros((seq_len, d_model), dtype=np.float32)
    pe[:, 0::2] = np.sin(position * div_term)
    pe[:, 1::2] = np.cos(position * div_term)
    return pe


_PE = _positional_encoding_np(S, D)

_mesh = plsc.VectorSubcoreMesh(core_axis_name="c", subcore_axis_name="s")


@functools.partial(
    pl.kernel,
    mesh=_mesh,
    compiler_params=pltpu.CompilerParams(use_tc_tiling_on_sc=False),
    out_type=jax.ShapeDtypeStruct((2 * S, B, D), jnp.float32),
    scratch_types=[
        pltpu.VMEM((S, BPW), jnp.int32),
        pltpu.VMEM((S, BPW), jnp.int32),
        pltpu.VMEM((S, D), jnp.float32),
        pltpu.VMEM((2, BPW, D), jnp.float32),
        pltpu.VMEM((2, BPW, D), jnp.float32),
        pltpu.SemaphoreType.DMA,
        pltpu.SemaphoreType.DMA,
        pltpu.SemaphoreType.DMA,
        pltpu.SemaphoreType.DMA,
        pltpu.SemaphoreType.DMA,
        pltpu.SemaphoreType.DMA,
        pltpu.SemaphoreType.DMA,
        pltpu.SemaphoreType.DMA,
    ],
)
def _embed_kernel(
    node_idx_hbm,
    edge_idx_hbm,
    node_table_hbm,
    edge_table_hbm,
    pe_hbm,
    out_hbm,
    idx_n,
    idx_e,
    pe_v,
    nbuf,
    ebuf,
    gn0,
    gn1,
    ge0,
    ge1,
    sn0,
    sn1,
    se0,
    se1,
):
    cid = lax.axis_index("c")
    sid = lax.axis_index("s")
    wid = sid * NC + cid
    base = wid * BPW

    gn = (gn0, gn1)
    ge = (ge0, ge1)
    sn = (sn0, sn1)
    se = (se0, se1)

    pltpu.sync_copy(node_idx_hbm.at[:, pl.ds(base, BPW)], idx_n)
    pltpu.sync_copy(edge_idx_hbm.at[:, pl.ds(base, BPW)], idx_e)
    pltpu.sync_copy(pe_hbm, pe_v)

    def start_gather(sp, k):
        pltpu.async_copy(node_table_hbm.at[idx_n.at[sp]], nbuf.at[k], gn[k])
        pltpu.async_copy(edge_table_hbm.at[idx_e.at[sp]], ebuf.at[k], ge[k])

    def wait_gather(sp, k):
        pltpu.make_async_copy(node_table_hbm.at[idx_n.at[sp]], nbuf.at[k], gn[k]).wait()
        pltpu.make_async_copy(edge_table_hbm.at[idx_e.at[sp]], ebuf.at[k], ge[k]).wait()

    def start_scatter(sp, k):
        pltpu.async_copy(nbuf.at[k], out_hbm.at[sp, pl.ds(base, BPW)], sn[k])
        pltpu.async_copy(ebuf.at[k], out_hbm.at[S + sp, pl.ds(base, BPW)], se[k])

    def wait_scatter(sp, k):
        pltpu.make_async_copy(nbuf.at[k], out_hbm.at[sp, pl.ds(base, BPW)], sn[k]).wait()
        pltpu.make_async_copy(
            ebuf.at[k], out_hbm.at[S + sp, pl.ds(base, BPW)], se[k]
        ).wait()

    def add_pe(sp, k):
        pv = [pe_v[sp, pl.ds(L * j, L)] for j in range(D // L)]

        def row(i, carry):
            for jr in range(4):
                r = i * 4 + jr
                for j in range(D // L):
                    sl = pl.ds(L * j, L)
                    nbuf[k, r, sl] = nbuf[k, r, sl] + pv[j]
                    ebuf[k, r, sl] = ebuf[k, r, sl] + pv[j]
            return carry

        lax.fori_loop(0, BPW // 4, row, None)

    start_gather(0, 0)

    def body(i, carry):
        s = i * 2
        @pl.when(s >= 1)
        def _():
            wait_scatter(s - 1, 1)

        start_gather(s + 1, 1)
        wait_gather(s, 0)
        add_pe(s, 0)
        start_scatter(s, 0)

        wait_scatter(s, 0)

        @pl.when(s + 2 < S)
        def _():
            start_gather(s + 2, 0)

        wait_gather(s + 1, 1)
        add_pe(s + 1, 1)
        start_scatter(s + 1, 1)
        return carry

    lax.fori_loop(0, S // 2, body, None)

    wait_scatter(S - 1, 1)


def kernel(node_idx, edge_idx, node_table, edge_table):
    pe = jnp.asarray(_PE)
    return _embed_kernel(
        node_idx.astype(jnp.int32),
        edge_idx.astype(jnp.int32),
        node_table.astype(jnp.float32),
        edge_table.astype(jnp.float32),
        pe,
    )
